# baseline (speedup 1.0000x reference)
import functools

import jax
import jax.numpy as jnp
from jax import lax
from jax.experimental import pallas as pl
from jax.experimental.pallas import tpu as pltpu
from jax.experimental.pallas import tpu_sc as plsc

N_NODES = 100000
N_EDGES = 1600000
NODE_FEAT = 128
NODE_EMB = 128
EDGE_EMB = 16
N_EDGE_TYPE = 16

NC = 2
NS = 16
NW = NC * NS
LANES = 16

ETILE = 128
N_ETILES = N_EDGES // ETILE
JT = 2
JIN = 8
PART = N_ETILES * JIN * ETILE

TILES_PER_CHUNK = 25
CHUNK = TILES_PER_CHUNK * ETILE
N_CHUNKS_TOTAL = N_ETILES // TILES_PER_CHUNK
CHUNK_PART = TILES_PER_CHUNK * JIN * ETILE
CHUNK_FLOATS = JT * CHUNK_PART
N_FULL = N_CHUNKS_TOTAL % NW
N_HI = -(-N_CHUNKS_TOTAL // NW)

_SC_MESH = plsc.VectorSubcoreMesh(
    core_axis_name="c", subcore_axis_name="s", num_cores=NC, num_subcores=NS
)


@functools.partial(
    pl.kernel,
    out_type=jax.ShapeDtypeStruct((N_EDGES * EDGE_EMB,), jnp.float32),
    mesh=_SC_MESH,
    scratch_types=[
        pltpu.VMEM((N_EDGE_TYPE, EDGE_EMB), jnp.float32),
        pltpu.VMEM((EDGE_EMB, EDGE_EMB), jnp.float32),
        pltpu.VMEM((EDGE_EMB,), jnp.float32),
        pltpu.VMEM((N_EDGE_TYPE, EDGE_EMB), jnp.float32),
        pltpu.VMEM((CHUNK,), jnp.int32),
        pltpu.VMEM((CHUNK,), jnp.int32),
        pltpu.VMEM((CHUNK_FLOATS,), jnp.float32),
        pltpu.VMEM((CHUNK_FLOATS,), jnp.float32),
        pltpu.SemaphoreType.DMA,
        pltpu.SemaphoreType.DMA,
        pltpu.SemaphoreType.DMA,
        pltpu.SemaphoreType.DMA,
    ],
    compiler_params=pltpu.CompilerParams(
        use_tc_tiling_on_sc=False,
        needs_layout_passes=False,
        disable_bounds_checks=True,
    ),
)
def _edge_path_sc(
    idx_hbm, embt_hbm, w2_hbm, b2_hbm, out_hbm,
    embt_v, w2_v, b2_v, table_v, idx0, idx1, rows0, rows1,
    sem_i0, sem_i1, sem_o0, sem_o1,
):
    wid = lax.axis_index("s") * NC + lax.axis_index("c")
    n_w = jnp.where(wid < N_FULL, N_HI, N_HI - 1)

    idx_bufs = (idx0, idx1)
    rows_bufs = (rows0, rows1)
    sems_i = (sem_i0, sem_i1)
    sems_o = (sem_o0, sem_o1)

    def chunk_id(i):
        return wid + NW * i

    def issue_idx(i, b):
        pltpu.async_copy(
            idx_hbm.at[pl.ds(chunk_id(i) * CHUNK, CHUNK)], idx_bufs[b], sems_i[b]
        )

    def wait_idx(b):
        pltpu.make_async_copy(
            idx_hbm.at[pl.ds(0, CHUNK)], idx_bufs[b], sems_i[b]
        ).wait()

    def issue_out(i, b):
        k = chunk_id(i)
        rb = rows_bufs[b]
        pltpu.async_copy(
            rb.at[pl.ds(0, CHUNK_PART)],
            out_hbm.at[pl.ds(k * CHUNK_PART, CHUNK_PART)],
            sems_o[b],
        )
        pltpu.async_copy(
            rb.at[pl.ds(CHUNK_PART, CHUNK_PART)],
            out_hbm.at[pl.ds(PART + k * CHUNK_PART, CHUNK_PART)],
            sems_o[b],
        )

    def drain_out(b):
        pltpu.make_async_copy(
            rows_bufs[b], out_hbm.at[pl.ds(0, CHUNK_FLOATS)], sems_o[b]
        ).wait()

    issue_idx(0, 0)
    issue_idx(1, 1)

    pltpu.sync_copy(embt_hbm, embt_v)
    pltpu.sync_copy(w2_hbm, w2_v)
    pltpu.sync_copy(b2_hbm, b2_v)
    b2_row = b2_v[...]
    for j in range(N_EDGE_TYPE):
        w_row = w2_v[j, :]
        acc = jnp.full((LANES,), 0.0, dtype=jnp.float32) + b2_row[j]
        for k in range(EDGE_EMB):
            acc = acc + w_row[k] * embt_v[k, :]
        table_v[j, :] = acc

    colsel = [jnp.full((LANES,), j, dtype=jnp.int32) for j in range(EDGE_EMB)]
    joff = [(j // JIN) * CHUNK_PART + (j % JIN) * ETILE for j in range(EDGE_EMB)]

    def compute_chunk(b):
        ib = idx_bufs[b]
        rb = rows_bufs[b]

        @plsc.parallel_loop(0, CHUNK // LANES, unroll=4)
        def g_body(g):
            idxv = ib[pl.ds(g * LANES, LANES)]
            base_g = (g >> 3) * (JIN * ETILE) + (g & 7) * LANES
            for j in range(EDGE_EMB):
                vals = plsc.load_gather(table_v, [colsel[j], idxv])
                rb[pl.ds(base_g + joff[j], LANES)] = vals

    def process(i, b):
        wait_idx(b)

        @pl.when(i > 1)
        def _():
            drain_out(b)

        compute_chunk(b)
        issue_out(i, b)

        @pl.when(i + 2 < n_w)
        def _():
            issue_idx(i + 2, b)

    def pair_body(i2, carry):
        for b in range(2):
            process(2 * i2 + b, b)
        return carry

    lax.fori_loop(0, n_w >> 1, pair_body, 0)

    @pl.when((n_w & 1) == 1)
    def _():
        process(n_w - 1, 0)

    drain_out(0)
    drain_out(1)


NODE_BLK = 5000


def _node_mlp_body(x_ref, w_ref, b_ref, o_ref):
    o_ref[...] = (
        jnp.dot(x_ref[...], w_ref[...], preferred_element_type=jnp.float32)
        + b_ref[...]
    )


def _node_mlp(x, w1t, b1):
    grid = N_NODES // NODE_BLK
    return pl.pallas_call(
        _node_mlp_body,
        grid=(grid,),
        in_specs=[
            pl.BlockSpec((NODE_BLK, NODE_FEAT), lambda i: (i, 0)),
            pl.BlockSpec((NODE_FEAT, NODE_EMB), lambda i: (0, 0)),
            pl.BlockSpec((1, NODE_EMB), lambda i: (0, 0)),
        ],
        out_specs=pl.BlockSpec((NODE_BLK, NODE_EMB), lambda i: (i, 0)),
        out_shape=jax.ShapeDtypeStruct((N_NODES, NODE_EMB), jnp.float32),
    )(x, w1t, b1)


def kernel(raw_node_features, raw_edge_features, W1, b1, emb_table, W2, b2):
    node_outputs = _node_mlp(raw_node_features, W1.T, b1.reshape(1, NODE_EMB))
    edge_flat = _edge_path_sc(raw_edge_features, emb_table.T, W2, b2)
    edge_outputs = (
        edge_flat.reshape(JT, N_ETILES, JIN, ETILE)
        .transpose(1, 3, 0, 2)
        .reshape(N_EDGES, EDGE_EMB)
    )
    return (node_outputs, edge_outputs)
